# Initial kernel scaffold; baseline (speedup 1.0000x reference)
import jax
import jax.numpy as jnp
from jax.experimental import pallas as pl


def kernel(g, x, batch, pre_w1, pre_b1, pre_w2, pre_b2, s_w, s_b, h_w, h_b, out_w, out_b, post_w1, post_b1, post_w2, post_b2, fin_w, fin_b):
    raise NotImplementedError("write your pallas kernel here")


# speedup vs baseline: 6.5388x; 6.5388x over previous
import functools

import jax
import jax.numpy as jnp
from jax.experimental import pallas as pl
from jax.experimental.pallas import tpu as pltpu

N = 10000
D_IN = 128
D = 32
S_DIM = 3
P_DIM = 64
K = 40
NB = 8

R = 256
C = 512
NP = 10240
NT = NP // R
BIG = 1e9
BIGI = 2147483647


def _elu(v):
    return jnp.where(v > 0, v, jnp.exp(v) - 1.0)


def _front_body(x_ref, w1_ref, b1_ref, w2_ref, b2_ref, sw_ref, sb_ref,
                hw_ref, hb_ref, h_ref, s_ref, f_ref):
    xb = x_ref[...]
    h1 = _elu(jnp.dot(xb, w1_ref[...], preferred_element_type=jnp.float32)
              + b1_ref[...])
    h2 = _elu(jnp.dot(h1, w2_ref[...], preferred_element_type=jnp.float32)
              + b2_ref[...])
    h_ref[...] = h2
    s_ref[...] = (jnp.dot(h2, sw_ref[...], preferred_element_type=jnp.float32)
                  + sb_ref[...])
    f_ref[...] = (jnp.dot(h2, hw_ref[...], preferred_element_type=jnp.float32)
                  + hb_ref[...])


def _knn_body(win_ref, s8_ref, s8t_ref, br_ref, bc_ref, feat_ref,
              idx_ref, mean_ref, max_ref, part_ref, d2_scr):
    t = pl.program_id(0)
    lo_t = win_ref[t, 0]
    hi_t = win_ref[t, 1]

    a = s8_ref[...]
    sqr = jnp.sum(a * a, axis=1, keepdims=True)
    br = br_ref[...]
    row_gid = (t * R
               + jax.lax.broadcasted_iota(jnp.int32, (R, 1), 0))
    lane_c = jax.lax.broadcasted_iota(jnp.int32, (R, C), 1)
    lane_k = jax.lax.broadcasted_iota(jnp.int32, (R, K), 1)

    def build(c, _):
        b = s8t_ref[:, pl.ds(c * C, C)]
        sqc = jnp.sum(b * b, axis=0, keepdims=True)
        cross = jax.lax.dot_general(
            a, b, (((1,), (0,)), ((), ())),
            preferred_element_type=jnp.float32)
        d2 = (sqr + sqc) - 2.0 * cross
        bc = bc_ref[:, pl.ds(c * C, C)]
        col_gid = c * C + lane_c
        valid = (br == bc) & (row_gid != col_gid)
        d2_scr[:, pl.ds(c * C, C)] = jnp.where(valid, d2, BIG)
        return 0

    jax.lax.fori_loop(lo_t, hi_t, build, 0)

    valid_row = (row_gid < N).astype(jnp.float32)

    def extract(k, carry):
        mean_acc, max_acc, idx_acc, d2s, ws = carry

        def scan1(c, mc):
            rmin, rarg = mc
            blk = d2_scr[:, pl.ds(c * C, C)]
            lmin = jnp.min(blk, axis=1, keepdims=True)
            lidx = jnp.min(
                jnp.where(blk == lmin, c * C + lane_c, BIGI),
                axis=1, keepdims=True)
            upd = lmin < rmin
            return (jnp.where(upd, lmin, rmin), jnp.where(upd, lidx, rarg))

        rmin, rarg = jax.lax.fori_loop(
            lo_t, hi_t, scan1,
            (jnp.full((R, 1), 2e9, jnp.float32),
             jnp.zeros((R, 1), jnp.int32)))

        def scan2(c, g):
            blk = d2_scr[:, pl.ds(c * C, C)]
            onehot = (c * C + lane_c) == rarg
            d2_scr[:, pl.ds(c * C, C)] = jnp.where(onehot, BIG, blk)
            fblk = feat_ref[pl.ds(c * C, C), :]
            return g + jax.lax.dot_general(
                onehot.astype(jnp.float32), fblk,
                (((1,), (0,)), ((), ())),
                preferred_element_type=jnp.float32)

        gath = jax.lax.fori_loop(lo_t, hi_t, scan2,
                                 jnp.zeros((R, P_DIM), jnp.float32))

        w = jnp.exp(-10.0 * rmin)
        msg = w * gath
        mean_acc = mean_acc + msg
        max_acc = jnp.maximum(max_acc, msg)
        idx_acc = jnp.where(lane_k == k, rarg, idx_acc)
        d2s = d2s + jnp.sum(valid_row * rmin, keepdims=True).reshape(1, 1)
        ws = ws + jnp.sum(valid_row * w, keepdims=True).reshape(1, 1)
        return (mean_acc, max_acc, idx_acc, d2s, ws)

    mean_acc, max_acc, idx_acc, d2s, ws = jax.lax.fori_loop(
        0, K, extract,
        (jnp.zeros((R, P_DIM), jnp.float32),
         jnp.full((R, P_DIM), -3e38, jnp.float32),
         jnp.zeros((R, K), jnp.int32),
         jnp.zeros((1, 1), jnp.float32),
         jnp.zeros((1, 1), jnp.float32)))

    idx_ref[...] = idx_acc
    mean_ref[...] = mean_acc * jnp.float32(1.0 / K)
    max_ref[...] = max_acc
    lane128 = jax.lax.broadcasted_iota(jnp.int32, (1, 128), 1)
    part_ref[0] = jnp.where(lane128 == 0, d2s,
                            jnp.where(lane128 == 1, ws, 0.0))


def _post_body(h_ref, mean_ref, max_ref, s8_ref, br_ref,
               ow_ref, ob_ref, p1_ref, pb1_ref, p2_ref, pb2_ref,
               z_ref, ssum_ref, smin_ref, smax_ref):
    t = pl.program_id(0)
    gcat = jnp.concatenate([h_ref[...], mean_ref[...], max_ref[...]], axis=1)
    gn = (jnp.dot(gcat, ow_ref[...], preferred_element_type=jnp.float32)
          + ob_ref[...])
    zcat = jnp.concatenate([gn, s8_ref[...]], axis=1)
    z1 = _elu(jnp.dot(zcat, p1_ref[...], preferred_element_type=jnp.float32)
              + pb1_ref[...])
    z = _elu(jnp.dot(z1, p2_ref[...], preferred_element_type=jnp.float32)
             + pb2_ref[...])
    z_ref[...] = z

    br = br_ref[...]
    seg_io = jax.lax.broadcasted_iota(jnp.int32, (NB, R), 0)
    onehot_t = (seg_io == br.reshape(1, R)).astype(jnp.float32)
    psum = jax.lax.dot_general(onehot_t, z, (((1,), (0,)), ((), ())),
                               preferred_element_type=jnp.float32)

    mask3 = (jax.lax.broadcasted_iota(jnp.int32, (NB, R, 1), 0)
             == br.reshape(1, R, 1))
    z3 = z.reshape(1, R, D)
    pmin = jnp.min(jnp.where(mask3, z3, 3e38), axis=1)
    pmax = jnp.max(jnp.where(mask3, z3, -3e38), axis=1)

    @pl.when(t == 0)
    def _():
        ssum_ref[...] = jnp.zeros((NB, D), jnp.float32)
        smin_ref[...] = jnp.full((NB, D), 3e38, jnp.float32)
        smax_ref[...] = jnp.full((NB, D), -3e38, jnp.float32)

    ssum_ref[...] += psum
    smin_ref[...] = jnp.minimum(smin_ref[...], pmin)
    smax_ref[...] = jnp.maximum(smax_ref[...], pmax)


def _fin_body(z_ref, br_ref, ssum_ref, smin_ref, smax_ref, cinv_ref,
              fw_ref, fb_ref, out_ref):
    br = br_ref[...]
    onehot = (br == jax.lax.broadcasted_iota(jnp.int32, (1, NB), 1)
              ).astype(jnp.float32)
    smean = ssum_ref[...] * cinv_ref[...]

    def pick(tab):
        return jnp.dot(onehot, tab, preferred_element_type=jnp.float32)

    zg = jnp.concatenate(
        [pick(smean), pick(smin_ref[...]), pick(smax_ref[...]), z_ref[...]],
        axis=1)
    out_ref[...] = _elu(
        jnp.dot(zg, fw_ref[...], preferred_element_type=jnp.float32)
        + fb_ref[...])


def _row_spec(w):
    return pl.BlockSpec((R, w), lambda t: (t, 0))


def _full_spec(shape):
    return pl.BlockSpec(shape, lambda t: (0, 0))


@jax.jit
def kernel(g, x, batch, pre_w1, pre_b1, pre_w2, pre_b2, s_w, s_b, h_w, h_b,
           out_w, out_b, post_w1, post_b1, post_w2, post_b2, fin_w, fin_b):
    f32 = jnp.float32
    xp = jnp.zeros((NP, D_IN), f32).at[:N].set(x)
    batch_p = jnp.concatenate(
        [batch, jnp.full((NP - N,), NB, jnp.int32)]).astype(jnp.int32)
    br2 = batch_p[:, None]
    bc2 = batch_p[None, :]

    bounds = jnp.searchsorted(batch_p, jnp.arange(NB + 2, dtype=jnp.int32),
                              side='left').astype(jnp.int32)
    row_starts = jnp.arange(NT, dtype=jnp.int32) * R
    b_first = batch_p[row_starts]
    b_last = batch_p[row_starts + R - 1]
    lo_t = bounds[b_first] // C
    hi_t = (bounds[b_last + 1] + C - 1) // C
    win = jnp.stack([lo_t, hi_t], axis=1).astype(jnp.int32)
    counts = jnp.maximum(
        (bounds[1:NB + 1] - bounds[:NB]).astype(f32), 1.0)
    cinv = (1.0 / counts)[:, None]

    s_w8 = jnp.zeros((D, 8), f32).at[:, :S_DIM].set(s_w)
    s_b8 = jnp.zeros((1, 8), f32).at[0, :S_DIM].set(s_b)
    p1_40 = jnp.zeros((D + 8, D), f32).at[:D + S_DIM].set(post_w1)

    h, s8, feat = pl.pallas_call(
        _front_body,
        grid=(NT,),
        in_specs=[_row_spec(D_IN),
                  _full_spec((D_IN, D)), _full_spec((1, D)),
                  _full_spec((D, D)), _full_spec((1, D)),
                  _full_spec((D, 8)), _full_spec((1, 8)),
                  _full_spec((D, P_DIM)), _full_spec((1, P_DIM))],
        out_specs=[_row_spec(D), _row_spec(8), _row_spec(P_DIM)],
        out_shape=[jax.ShapeDtypeStruct((NP, D), f32),
                   jax.ShapeDtypeStruct((NP, 8), f32),
                   jax.ShapeDtypeStruct((NP, P_DIM), f32)],
    )(xp, pre_w1, pre_b1[None, :], pre_w2, pre_b2[None, :],
      s_w8, s_b8, h_w, h_b[None, :])

    s8t = s8.T

    idx, mean_agg, max_agg, parts = pl.pallas_call(
        _knn_body,
        grid=(NT,),
        in_specs=[pl.BlockSpec(memory_space=pltpu.SMEM),
                  _row_spec(8),
                  _full_spec((8, NP)),
                  pl.BlockSpec((R, 1), lambda t: (t, 0)),
                  _full_spec((1, NP)),
                  _full_spec((NP, P_DIM))],
        out_specs=[pl.BlockSpec((R, K), lambda t: (t, 0)),
                   _row_spec(P_DIM), _row_spec(P_DIM),
                   pl.BlockSpec((1, 1, 128), lambda t: (t, 0, 0))],
        out_shape=[jax.ShapeDtypeStruct((NP, K), jnp.int32),
                   jax.ShapeDtypeStruct((NP, P_DIM), f32),
                   jax.ShapeDtypeStruct((NP, P_DIM), f32),
                   jax.ShapeDtypeStruct((NT, 1, 128), f32)],
        scratch_shapes=[pltpu.VMEM((R, NP), f32)],
    )(win, s8, s8t, br2, bc2, feat)

    z, ssum, smin, smax = pl.pallas_call(
        _post_body,
        grid=(NT,),
        in_specs=[_row_spec(D), _row_spec(P_DIM), _row_spec(P_DIM),
                  _row_spec(8),
                  pl.BlockSpec((R, 1), lambda t: (t, 0)),
                  _full_spec((D + 2 * P_DIM, D)), _full_spec((1, D)),
                  _full_spec((D + 8, D)), _full_spec((1, D)),
                  _full_spec((D, D)), _full_spec((1, D))],
        out_specs=[_row_spec(D),
                   _full_spec((NB, D)), _full_spec((NB, D)),
                   _full_spec((NB, D))],
        out_shape=[jax.ShapeDtypeStruct((NP, D), f32),
                   jax.ShapeDtypeStruct((NB, D), f32),
                   jax.ShapeDtypeStruct((NB, D), f32),
                   jax.ShapeDtypeStruct((NB, D), f32)],
    )(h, mean_agg, max_agg, s8, br2,
      out_w, out_b[None, :], p1_40, post_b1[None, :],
      post_w2, post_b2[None, :])

    out = pl.pallas_call(
        _fin_body,
        grid=(NT,),
        in_specs=[_row_spec(D),
                  pl.BlockSpec((R, 1), lambda t: (t, 0)),
                  _full_spec((NB, D)), _full_spec((NB, D)),
                  _full_spec((NB, D)), _full_spec((NB, 1)),
                  _full_spec((4 * D, D)), _full_spec((1, D))],
        out_specs=_row_spec(D),
        out_shape=jax.ShapeDtypeStruct((NP, D), f32),
    )(z, br2, ssum, smin, smax, cinv, fin_w, fin_b[None, :])

    graph = jnp.stack([idx[:N].reshape(-1),
                       jnp.repeat(jnp.arange(N, dtype=jnp.int32), K)], axis=0)
    loss_reg = jnp.sum(parts[:, 0, 0]) / jnp.float32(N * K)
    ll_r = jnp.sum(parts[:, 0, 1]) / jnp.float32(N * K)
    return (out[:N], graph, loss_reg, ll_r)


# speedup vs baseline: 8.6052x; 1.3160x over previous
import functools

import jax
import jax.numpy as jnp
from jax.experimental import pallas as pl
from jax.experimental.pallas import tpu as pltpu

N = 10000
D_IN = 128
D = 32
S_DIM = 3
P_DIM = 64
K = 40
NB = 8

R = 256
C = 512
NP = 10240
NT = NP // R
BIG = 1e9
BIGI = 2147483647


def _elu(v):
    return jnp.where(v > 0, v, jnp.exp(v) - 1.0)


def _front_body(x_ref, w1_ref, b1_ref, w2_ref, b2_ref, sw_ref, sb_ref,
                hw_ref, hb_ref, h_ref, s_ref, f_ref):
    xb = x_ref[...]
    h1 = _elu(jnp.dot(xb, w1_ref[...], preferred_element_type=jnp.float32)
              + b1_ref[...])
    h2 = _elu(jnp.dot(h1, w2_ref[...], preferred_element_type=jnp.float32)
              + b2_ref[...])
    h_ref[...] = h2
    s_ref[...] = (jnp.dot(h2, sw_ref[...], preferred_element_type=jnp.float32)
                  + sb_ref[...])
    f_ref[...] = (jnp.dot(h2, hw_ref[...], preferred_element_type=jnp.float32)
                  + hb_ref[...])


def _knn_body(win_ref, s8_ref, s8t_ref, br_ref, bc_ref, feat_ref,
              idx_ref, mean_ref, max_ref, part_ref, d2_scr):
    t = pl.program_id(0)
    lo_t = win_ref[t, 0]
    hi_t = win_ref[t, 1]

    a = s8_ref[...]
    sqr = jnp.sum(a * a, axis=1, keepdims=True)
    br = br_ref[...]
    row_gid = (t * R
               + jax.lax.broadcasted_iota(jnp.int32, (R, 1), 0))
    lane_c = jax.lax.broadcasted_iota(jnp.int32, (R, C), 1)
    lane_k = jax.lax.broadcasted_iota(jnp.int32, (R, K), 1)

    def build(c, mc):
        rmin, rarg = mc
        b = s8t_ref[:, pl.ds(c * C, C)]
        sqc = jnp.sum(b * b, axis=0, keepdims=True)
        cross = jax.lax.dot_general(
            a, b, (((1,), (0,)), ((), ())),
            preferred_element_type=jnp.float32)
        d2 = (sqr + sqc) - 2.0 * cross
        bc = bc_ref[:, pl.ds(c * C, C)]
        col_gid = c * C + lane_c
        valid = (br == bc) & (row_gid != col_gid)
        d2 = jnp.where(valid, d2, BIG)
        d2_scr[:, pl.ds(c * C, C)] = d2
        lmin = jnp.min(d2, axis=1, keepdims=True)
        lidx = jnp.min(jnp.where(d2 == lmin, col_gid, BIGI),
                       axis=1, keepdims=True)
        upd = lmin < rmin
        return (jnp.where(upd, lmin, rmin), jnp.where(upd, lidx, rarg))

    rmin, rarg = jax.lax.fori_loop(
        lo_t, hi_t, build,
        (jnp.full((R, 1), 2e9, jnp.float32),
         jnp.zeros((R, 1), jnp.int32)))

    valid_row = (row_gid < N).astype(jnp.float32)

    def extract(k, carry):
        mean_acc, max_acc, idx_acc, d2s, ws, rmin, rarg = carry

        def scan(c, st):
            g, nmin, narg = st
            col_gid = c * C + lane_c
            blk = d2_scr[:, pl.ds(c * C, C)]
            onehot = col_gid == rarg
            blk = jnp.where(onehot, BIG, blk)
            d2_scr[:, pl.ds(c * C, C)] = blk
            fblk = feat_ref[pl.ds(c * C, C), :]
            g = g + jax.lax.dot_general(
                onehot.astype(jnp.float32), fblk,
                (((1,), (0,)), ((), ())),
                preferred_element_type=jnp.float32)
            lmin = jnp.min(blk, axis=1, keepdims=True)
            lidx = jnp.min(jnp.where(blk == lmin, col_gid, BIGI),
                           axis=1, keepdims=True)
            upd = lmin < nmin
            return (g, jnp.where(upd, lmin, nmin), jnp.where(upd, lidx, narg))

        gath, nmin, narg = jax.lax.fori_loop(
            lo_t, hi_t, scan,
            (jnp.zeros((R, P_DIM), jnp.float32),
             jnp.full((R, 1), 2e9, jnp.float32),
             jnp.zeros((R, 1), jnp.int32)))

        w = jnp.exp(-10.0 * rmin)
        msg = w * gath
        mean_acc = mean_acc + msg
        max_acc = jnp.maximum(max_acc, msg)
        idx_acc = jnp.where(lane_k == k, rarg, idx_acc)
        d2s = d2s + jnp.sum(valid_row * rmin, keepdims=True).reshape(1, 1)
        ws = ws + jnp.sum(valid_row * w, keepdims=True).reshape(1, 1)
        return (mean_acc, max_acc, idx_acc, d2s, ws, nmin, narg)

    mean_acc, max_acc, idx_acc, d2s, ws, _, _ = jax.lax.fori_loop(
        0, K, extract,
        (jnp.zeros((R, P_DIM), jnp.float32),
         jnp.full((R, P_DIM), -3e38, jnp.float32),
         jnp.zeros((R, K), jnp.int32),
         jnp.zeros((1, 1), jnp.float32),
         jnp.zeros((1, 1), jnp.float32),
         rmin, rarg))

    idx_ref[...] = idx_acc
    mean_ref[...] = mean_acc * jnp.float32(1.0 / K)
    max_ref[...] = max_acc
    lane128 = jax.lax.broadcasted_iota(jnp.int32, (1, 128), 1)
    part_ref[0] = jnp.where(lane128 == 0, d2s,
                            jnp.where(lane128 == 1, ws, 0.0))


def _post_body(h_ref, mean_ref, max_ref, s8_ref, br_ref,
               ow_ref, ob_ref, p1_ref, pb1_ref, p2_ref, pb2_ref,
               z_ref, ssum_ref, smin_ref, smax_ref):
    t = pl.program_id(0)
    gcat = jnp.concatenate([h_ref[...], mean_ref[...], max_ref[...]], axis=1)
    gn = (jnp.dot(gcat, ow_ref[...], preferred_element_type=jnp.float32)
          + ob_ref[...])
    zcat = jnp.concatenate([gn, s8_ref[...]], axis=1)
    z1 = _elu(jnp.dot(zcat, p1_ref[...], preferred_element_type=jnp.float32)
              + pb1_ref[...])
    z = _elu(jnp.dot(z1, p2_ref[...], preferred_element_type=jnp.float32)
             + pb2_ref[...])
    z_ref[...] = z

    br = br_ref[...]
    seg_io = jax.lax.broadcasted_iota(jnp.int32, (NB, R), 0)
    onehot_t = (seg_io == br.reshape(1, R)).astype(jnp.float32)
    psum = jax.lax.dot_general(onehot_t, z, (((1,), (0,)), ((), ())),
                               preferred_element_type=jnp.float32)

    mask3 = (jax.lax.broadcasted_iota(jnp.int32, (NB, R, 1), 0)
             == br.reshape(1, R, 1))
    z3 = z.reshape(1, R, D)
    pmin = jnp.min(jnp.where(mask3, z3, 3e38), axis=1)
    pmax = jnp.max(jnp.where(mask3, z3, -3e38), axis=1)

    @pl.when(t == 0)
    def _():
        ssum_ref[...] = jnp.zeros((NB, D), jnp.float32)
        smin_ref[...] = jnp.full((NB, D), 3e38, jnp.float32)
        smax_ref[...] = jnp.full((NB, D), -3e38, jnp.float32)

    ssum_ref[...] += psum
    smin_ref[...] = jnp.minimum(smin_ref[...], pmin)
    smax_ref[...] = jnp.maximum(smax_ref[...], pmax)


def _fin_body(z_ref, br_ref, ssum_ref, smin_ref, smax_ref, cinv_ref,
              fw_ref, fb_ref, out_ref):
    br = br_ref[...]
    onehot = (br == jax.lax.broadcasted_iota(jnp.int32, (1, NB), 1)
              ).astype(jnp.float32)
    smean = ssum_ref[...] * cinv_ref[...]

    def pick(tab):
        return jnp.dot(onehot, tab, preferred_element_type=jnp.float32)

    zg = jnp.concatenate(
        [pick(smean), pick(smin_ref[...]), pick(smax_ref[...]), z_ref[...]],
        axis=1)
    out_ref[...] = _elu(
        jnp.dot(zg, fw_ref[...], preferred_element_type=jnp.float32)
        + fb_ref[...])


def _row_spec(w):
    return pl.BlockSpec((R, w), lambda t: (t, 0))


def _full_spec(shape):
    return pl.BlockSpec(shape, lambda t: (0, 0))


@jax.jit
def kernel(g, x, batch, pre_w1, pre_b1, pre_w2, pre_b2, s_w, s_b, h_w, h_b,
           out_w, out_b, post_w1, post_b1, post_w2, post_b2, fin_w, fin_b):
    f32 = jnp.float32
    xp = jnp.zeros((NP, D_IN), f32).at[:N].set(x)
    batch_p = jnp.concatenate(
        [batch, jnp.full((NP - N,), NB, jnp.int32)]).astype(jnp.int32)
    br2 = batch_p[:, None]
    bc2 = batch_p[None, :]

    bounds = jnp.searchsorted(batch_p, jnp.arange(NB + 2, dtype=jnp.int32),
                              side='left').astype(jnp.int32)
    row_starts = jnp.arange(NT, dtype=jnp.int32) * R
    b_first = batch_p[row_starts]
    b_last = batch_p[row_starts + R - 1]
    lo_t = bounds[b_first] // C
    hi_t = (bounds[b_last + 1] + C - 1) // C
    win = jnp.stack([lo_t, hi_t], axis=1).astype(jnp.int32)
    counts = jnp.maximum(
        (bounds[1:NB + 1] - bounds[:NB]).astype(f32), 1.0)
    cinv = (1.0 / counts)[:, None]

    s_w8 = jnp.zeros((D, 8), f32).at[:, :S_DIM].set(s_w)
    s_b8 = jnp.zeros((1, 8), f32).at[0, :S_DIM].set(s_b)
    p1_40 = jnp.zeros((D + 8, D), f32).at[:D + S_DIM].set(post_w1)

    h, s8, feat = pl.pallas_call(
        _front_body,
        grid=(NT,),
        in_specs=[_row_spec(D_IN),
                  _full_spec((D_IN, D)), _full_spec((1, D)),
                  _full_spec((D, D)), _full_spec((1, D)),
                  _full_spec((D, 8)), _full_spec((1, 8)),
                  _full_spec((D, P_DIM)), _full_spec((1, P_DIM))],
        out_specs=[_row_spec(D), _row_spec(8), _row_spec(P_DIM)],
        out_shape=[jax.ShapeDtypeStruct((NP, D), f32),
                   jax.ShapeDtypeStruct((NP, 8), f32),
                   jax.ShapeDtypeStruct((NP, P_DIM), f32)],
    )(xp, pre_w1, pre_b1[None, :], pre_w2, pre_b2[None, :],
      s_w8, s_b8, h_w, h_b[None, :])

    s8t = s8.T

    idx, mean_agg, max_agg, parts = pl.pallas_call(
        _knn_body,
        grid=(NT,),
        in_specs=[pl.BlockSpec(memory_space=pltpu.SMEM),
                  _row_spec(8),
                  _full_spec((8, NP)),
                  pl.BlockSpec((R, 1), lambda t: (t, 0)),
                  _full_spec((1, NP)),
                  _full_spec((NP, P_DIM))],
        out_specs=[pl.BlockSpec((R, K), lambda t: (t, 0)),
                   _row_spec(P_DIM), _row_spec(P_DIM),
                   pl.BlockSpec((1, 1, 128), lambda t: (t, 0, 0))],
        out_shape=[jax.ShapeDtypeStruct((NP, K), jnp.int32),
                   jax.ShapeDtypeStruct((NP, P_DIM), f32),
                   jax.ShapeDtypeStruct((NP, P_DIM), f32),
                   jax.ShapeDtypeStruct((NT, 1, 128), f32)],
        scratch_shapes=[pltpu.VMEM((R, NP), f32)],
    )(win, s8, s8t, br2, bc2, feat)

    z, ssum, smin, smax = pl.pallas_call(
        _post_body,
        grid=(NT,),
        in_specs=[_row_spec(D), _row_spec(P_DIM), _row_spec(P_DIM),
                  _row_spec(8),
                  pl.BlockSpec((R, 1), lambda t: (t, 0)),
                  _full_spec((D + 2 * P_DIM, D)), _full_spec((1, D)),
                  _full_spec((D + 8, D)), _full_spec((1, D)),
                  _full_spec((D, D)), _full_spec((1, D))],
        out_specs=[_row_spec(D),
                   _full_spec((NB, D)), _full_spec((NB, D)),
                   _full_spec((NB, D))],
        out_shape=[jax.ShapeDtypeStruct((NP, D), f32),
                   jax.ShapeDtypeStruct((NB, D), f32),
                   jax.ShapeDtypeStruct((NB, D), f32),
                   jax.ShapeDtypeStruct((NB, D), f32)],
    )(h, mean_agg, max_agg, s8, br2,
      out_w, out_b[None, :], p1_40, post_b1[None, :],
      post_w2, post_b2[None, :])

    out = pl.pallas_call(
        _fin_body,
        grid=(NT,),
        in_specs=[_row_spec(D),
                  pl.BlockSpec((R, 1), lambda t: (t, 0)),
                  _full_spec((NB, D)), _full_spec((NB, D)),
                  _full_spec((NB, D)), _full_spec((NB, 1)),
                  _full_spec((4 * D, D)), _full_spec((1, D))],
        out_specs=_row_spec(D),
        out_shape=jax.ShapeDtypeStruct((NP, D), f32),
    )(z, br2, ssum, smin, smax, cinv, fin_w, fin_b[None, :])

    graph = jnp.stack([idx[:N].reshape(-1),
                       jnp.repeat(jnp.arange(N, dtype=jnp.int32), K)], axis=0)
    loss_reg = jnp.sum(parts[:, 0, 0]) / jnp.float32(N * K)
    ll_r = jnp.sum(parts[:, 0, 1]) / jnp.float32(N * K)
    return (out[:N], graph, loss_reg, ll_r)
